# baseline (speedup 1.0000x reference)
import functools

import jax
import jax.numpy as jnp
from jax import lax
from jax.experimental import pallas as pl
from jax.experimental.pallas import tpu as pltpu
from jax.experimental.pallas import tpu_sc as plsc

YEAR_LO_CONST = 1919
YEAR_IDX_MAX = 81
YEAR_ROWS = 83

B = 16384
ITEM_D = 64
YEAR_D = 16


def _sc_item_gather(item_table, item_ids):
    info = plsc.get_sparse_core_info()
    nc, ns = info.num_cores, info.num_subcores
    nw = nc * ns
    b_per_w = B // nw
    quarter = b_per_w // 4
    chunk = 16

    mesh = plsc.VectorSubcoreMesh(core_axis_name="c", subcore_axis_name="s")

    @functools.partial(
        pl.kernel,
        mesh=mesh,
        out_type=jax.ShapeDtypeStruct((8 * B, 2 * ITEM_D), jnp.bfloat16),
        scratch_types=[
            pltpu.VMEM_SHARED((nw, b_per_w), jnp.int32),
            pltpu.SMEM((b_per_w,), jnp.int32),
            pltpu.VMEM((8 * quarter, 2 * ITEM_D), jnp.bfloat16),
            pltpu.SemaphoreType.DMA,
        ],
    )
    def sc_kernel(tab_hbm, ids_hbm, out_hbm, ids_sh, ids_s, rows_v, sem):
        wid = lax.axis_index("s") * nc + lax.axis_index("c")
        base = wid * b_per_w
        pltpu.sync_copy(ids_hbm.at[pl.ds(base, b_per_w)], ids_sh.at[wid])
        pltpu.sync_copy(ids_sh.at[wid], ids_s)

        for h in range(4):
            prev = []
            for c in range(quarter // chunk):
                cur = []
                for t in range(chunk):
                    jj = c * chunk + t
                    idx = ids_s[h * quarter + jj]
                    g8 = (idx // 16) * 8
                    cur.append(pltpu.async_copy(
                        tab_hbm.at[pl.ds(g8, 8)],
                        rows_v.at[pl.ds(8 * jj, 8)], sem))
                for cp in prev:
                    cp.wait()
                prev = cur
            for cp in prev:
                cp.wait()
            pltpu.sync_copy(
                rows_v,
                out_hbm.at[pl.ds(8 * (base + h * quarter), 8 * quarter)])

    return sc_kernel(item_table, item_ids)


def _tc_body(item_ref, sel_ref, par_ref, genre_ref, year_ref, title_ref,
             gwt_ref, gb_ref, ytabt_ref, w1t_ref, b1_ref, w2_ref, b2_ref,
             out_ref):
    genre = jnp.maximum(
        lax.dot_general(gwt_ref[...], genre_ref[...], (((1,), (0,)), ((), ())),
                        preferred_element_type=jnp.float32) + gb_ref[...], 0.0)
    yidx = jnp.clip(year_ref[...] - YEAR_LO_CONST, 0, YEAR_IDX_MAX)
    onehot = (yidx == lax.broadcasted_iota(
        jnp.int32, (YEAR_ROWS, year_ref.shape[1]), 0)).astype(jnp.float32)
    year = lax.dot_general(ytabt_ref[...], onehot, (((1,), (0,)), ((), ())),
                           preferred_element_type=jnp.float32)
    h = jnp.maximum(
        lax.dot_general(w1t_ref[...], title_ref[...], (((1,), (1,)), ((), ())),
                        preferred_element_type=jnp.float32) + b1_ref[...], 0.0)
    text = lax.dot_general(w2_ref[...], h, (((0,), (0,)), ((), ())),
                           preferred_element_type=jnp.float32) + b2_ref[...]
    sel = sel_ref[...]
    par = par_ref[...]
    v = item_ref[...]
    bb = sel.shape[0]
    left = v[:, :ITEM_D].reshape(bb, 8, ITEM_D)
    right = v[:, ITEM_D:].reshape(bb, 8, ITEM_D)
    pickl = left[:, 0, :]
    pickr = right[:, 0, :]
    for k in range(1, 8):
        pickl = jnp.where(sel == k, left[:, k, :], pickl)
        pickr = jnp.where(sel == k, right[:, k, :], pickr)
    item = jnp.where(par == 1, pickr, pickl).astype(jnp.float32).T
    out_ref[...] = jnp.concatenate([item, genre, year, text], axis=0)


def _tc_encoder_t(item_grp, sel_2d, par_2d, genre_t, years_2d, title,
                  gw_t, genre_b, year_table_t, w1_t, b1, w2, b2,
                  block_b=1024):
    grid = (B // block_b,)
    text_in = title.shape[1]
    h_d = w1_t.shape[0]
    text_d = w2.shape[1]
    genre_in = genre_t.shape[0]
    genre_d = gw_t.shape[0]
    out_d = ITEM_D + genre_d + YEAR_D + text_d

    def col_block(d):
        return pl.BlockSpec((d, block_b), lambda i: (0, i))

    def full_block(r, c):
        return pl.BlockSpec((r, c), lambda i: (0, 0))

    return pl.pallas_call(
        _tc_body,
        grid=grid,
        in_specs=[
            pl.BlockSpec((8 * block_b, 2 * ITEM_D), lambda i: (i, 0)),
            pl.BlockSpec((block_b, 1), lambda i: (i, 0)),
            pl.BlockSpec((block_b, 1), lambda i: (i, 0)),
            col_block(genre_in),
            col_block(1),
            pl.BlockSpec((block_b, text_in), lambda i: (i, 0)),
            full_block(genre_d, genre_in),
            full_block(genre_d, 1),
            full_block(YEAR_D, YEAR_ROWS),
            full_block(h_d, text_in),
            full_block(h_d, 1),
            full_block(h_d, text_d),
            full_block(text_d, 1),
        ],
        out_specs=col_block(out_d),
        out_shape=jax.ShapeDtypeStruct((out_d, B), jnp.float32),
        compiler_params=pltpu.CompilerParams(
            dimension_semantics=("arbitrary",)),
    )(item_grp, sel_2d, par_2d, genre_t, years_2d, title,
      gw_t, genre_b.reshape(-1, 1), year_table_t,
      w1_t, b1.reshape(-1, 1), w2, b2.reshape(-1, 1))


def kernel(item_ids, genre_vectors, release_years, title_embeddings,
           item_table, genre_W, genre_b, year_table,
           text_W1, text_b1, text_W2, text_b2):
    item_ids = item_ids.astype(jnp.int32)
    release_years = release_years.astype(jnp.int32)
    table_pairs16 = item_table.astype(jnp.bfloat16).reshape(
        item_table.shape[0] // 2, 2 * ITEM_D)
    item_grp = _sc_item_gather(table_pairs16, item_ids)
    sel_2d = (jnp.remainder(item_ids, 16) // 2).reshape(B, 1)
    par_2d = jnp.remainder(item_ids, 2).reshape(B, 1)
    out_t = _tc_encoder_t(item_grp, sel_2d, par_2d,
                          genre_vectors.astype(jnp.float32).T,
                          release_years.reshape(1, B), title_embeddings,
                          genre_W.T, genre_b, year_table.T,
                          text_W1.T, text_b1, text_W2, text_b2)
    return out_t.T


# speedup vs baseline: 2.0553x; 2.0553x over previous
import functools

import jax
import jax.numpy as jnp
from jax import lax
from jax.experimental import pallas as pl
from jax.experimental.pallas import tpu as pltpu
from jax.experimental.pallas import tpu_sc as plsc

YEAR_LO_CONST = 1919
YEAR_IDX_MAX = 81
YEAR_ROWS = 83

B = 16384
ITEM_D = 64
YEAR_D = 16


def _sc_item_gather(item_table, item_ids):
    info = plsc.get_sparse_core_info()
    nc, ns = info.num_cores, info.num_subcores
    nw = nc * ns
    b_per_w = B // nw
    chunk = 16
    n_chunks = b_per_w // chunk

    mesh = plsc.VectorSubcoreMesh(core_axis_name="c", subcore_axis_name="s")

    @functools.partial(
        pl.kernel,
        mesh=mesh,
        out_type=jax.ShapeDtypeStruct((B, ITEM_D), jnp.float32),
        scratch_types=[
            pltpu.VMEM_SHARED((nw, b_per_w), jnp.int32),
            pltpu.SMEM((b_per_w,), jnp.int32),
            pltpu.VMEM((b_per_w, ITEM_D), jnp.float32),
            pltpu.SemaphoreType.DMA,
        ],
    )
    def sc_kernel(tab_hbm, ids_hbm, out_hbm, ids_sh, ids_s, rows_v, sem):
        wid = lax.axis_index("s") * nc + lax.axis_index("c")
        base = wid * b_per_w
        pltpu.sync_copy(ids_hbm.at[pl.ds(base, b_per_w)], ids_sh.at[wid])
        pltpu.sync_copy(ids_sh.at[wid], ids_s)

        prev = []
        for c in range(n_chunks):
            cur = []
            for t in range(chunk):
                j = c * chunk + t
                idx = ids_s[j]
                cur.append(pltpu.async_copy(
                    tab_hbm.at[pl.ds(idx, 1)], rows_v.at[pl.ds(j, 1)], sem))
            for cp in prev:
                cp.wait()
            prev = cur
        for cp in prev:
            cp.wait()
        pltpu.sync_copy(rows_v, out_hbm.at[pl.ds(base, b_per_w)])

    return sc_kernel(item_table, item_ids)


def _tc_body(item_ref, genre_ref, year_ref, title_ref,
             gwt_ref, gb_ref, ytabt_ref, w1t_ref, b1_ref, w2_ref, b2_ref,
             out_ref):
    genre = jnp.maximum(
        lax.dot_general(gwt_ref[...], genre_ref[...], (((1,), (0,)), ((), ())),
                        preferred_element_type=jnp.float32) + gb_ref[...], 0.0)
    yidx = jnp.clip(year_ref[...] - YEAR_LO_CONST, 0, YEAR_IDX_MAX)
    onehot = (yidx == lax.broadcasted_iota(
        jnp.int32, (YEAR_ROWS, year_ref.shape[1]), 0)).astype(jnp.float32)
    year = lax.dot_general(ytabt_ref[...], onehot, (((1,), (0,)), ((), ())),
                           preferred_element_type=jnp.float32)
    h = jnp.maximum(
        lax.dot_general(w1t_ref[...], title_ref[...], (((1,), (1,)), ((), ())),
                        preferred_element_type=jnp.float32) + b1_ref[...], 0.0)
    text = lax.dot_general(w2_ref[...], h, (((0,), (0,)), ((), ())),
                           preferred_element_type=jnp.float32) + b2_ref[...]
    item = item_ref[...].T
    out_ref[...] = jnp.concatenate([item, genre, year, text], axis=0)


def _tc_encoder_t(item_emb, genre_t, years_2d, title, gw_t, genre_b,
                  year_table_t, w1_t, b1, w2, b2, block_b=2048):
    grid = (B // block_b,)
    text_in = title.shape[1]
    h_d = w1_t.shape[0]
    text_d = w2.shape[1]
    genre_in = genre_t.shape[0]
    genre_d = gw_t.shape[0]
    out_d = ITEM_D + genre_d + YEAR_D + text_d

    def col_block(d):
        return pl.BlockSpec((d, block_b), lambda i: (0, i))

    def full_block(r, c):
        return pl.BlockSpec((r, c), lambda i: (0, 0))

    return pl.pallas_call(
        _tc_body,
        grid=grid,
        in_specs=[
            pl.BlockSpec((block_b, ITEM_D), lambda i: (i, 0)),
            col_block(genre_in),
            col_block(1),
            pl.BlockSpec((block_b, text_in), lambda i: (i, 0)),
            full_block(genre_d, genre_in),
            full_block(genre_d, 1),
            full_block(YEAR_D, YEAR_ROWS),
            full_block(h_d, text_in),
            full_block(h_d, 1),
            full_block(h_d, text_d),
            full_block(text_d, 1),
        ],
        out_specs=col_block(out_d),
        out_shape=jax.ShapeDtypeStruct((out_d, B), jnp.float32),
        compiler_params=pltpu.CompilerParams(
            dimension_semantics=("arbitrary",)),
    )(item_emb, genre_t, years_2d, title,
      gw_t, genre_b.reshape(-1, 1), year_table_t,
      w1_t, b1.reshape(-1, 1), w2, b2.reshape(-1, 1))


def kernel(item_ids, genre_vectors, release_years, title_embeddings,
           item_table, genre_W, genre_b, year_table,
           text_W1, text_b1, text_W2, text_b2):
    item_ids = item_ids.astype(jnp.int32)
    release_years = release_years.astype(jnp.int32)
    item_emb = _sc_item_gather(item_table, item_ids)
    out_t = _tc_encoder_t(item_emb, genre_vectors.astype(jnp.float32).T,
                          release_years.reshape(1, B), title_embeddings,
                          genre_W.T, genre_b, year_table.T,
                          text_W1.T, text_b1, text_W2, text_b2)
    return out_t.T
